# baseline (speedup 1.0000x reference)
import jax
import jax.numpy as jnp
from jax.experimental import pallas as pl
from jax.experimental.pallas import tpu as pltpu

_K = 1024
_D = 64
_Q = 2


def _dot(a, b, dims):
    return jax.lax.dot_general(a, b, (dims, ((), ())),
                               preferred_element_type=jnp.float32,
                               precision=jax.lax.Precision.DEFAULT)


def _rvq_kernel(x_ref, cb2_ref, cb_ref, sp_ref, cn_ref,
                out_ref, idx0_ref, idx1_ref, loss_ref):
    x = jnp.transpose(x_ref[0], (1, 0))
    tb = x.shape[0]
    lane_iota = jax.lax.broadcasted_iota(jnp.int32, (1, 128), 1)
    residual = x
    quant_out = jnp.zeros_like(x)
    losses = []
    idx_refs = (idx0_ref, idx1_ref)
    for q in range(_Q):
        cnorm = cn_ref[q]
        rnorm = jnp.sum(residual * residual, axis=1, keepdims=True)
        prod2 = _dot(residual, cb2_ref[q], ((1,), (1,)))
        d = (rnorm - prod2) + cnorm[None, :]
        dmin = jnp.min(d, axis=1, keepdims=True)
        key = None
        for c in range(_K // 128):
            dc = jax.lax.slice_in_dim(d, c * 128, (c + 1) * 128, axis=1)
            kc = jnp.where(dc == dmin, lane_iota + (128 * c), _K)
            key = kc if key is None else jnp.minimum(key, kc)
        idxc = jnp.min(key, axis=1, keepdims=True)
        idx_refs[q][0, 0] = idxc[:, 0]
        onehot = jnp.concatenate(
            [(lane_iota + (128 * c) == idxc).astype(jnp.float32)
             for c in range(_K // 128)], axis=1)
        if q == 0:
            quant = ((_dot(onehot, sp_ref[0], ((1,), (0,)))
                      + _dot(onehot, sp_ref[1], ((1,), (0,))))
                     + _dot(onehot, sp_ref[2], ((1,), (0,))))
        else:
            quant = _dot(onehot, cb_ref[q], ((1,), (0,)))
        diff = quant - residual
        losses.append(jnp.sum(diff * diff))
        quant_out = quant_out + (residual + (quant - residual))
        residual = residual - quant
    out_ref[0] = jnp.transpose(quant_out, (1, 0))
    loss_ref[0] = jnp.stack([jnp.broadcast_to(l, (128,)) for l in losses])


def _split3(cb):
    mask = jnp.int32(-65536)
    hi = jnp.bitwise_and(cb.view(jnp.int32), mask).view(jnp.float32)
    r = cb - hi
    mid = jnp.bitwise_and(r.view(jnp.int32), mask).view(jnp.float32)
    lo = r - mid
    return jnp.stack([hi, mid, lo])


def _rvq(x, codebooks, tb, interpret=False):
    b, d_, t = x.shape
    g = t // tb
    cnorm = jnp.sum(codebooks * codebooks, axis=-1)
    out, idx0, idx1, lossp = pl.pallas_call(
        _rvq_kernel,
        grid=(b, g),
        in_specs=[
            pl.BlockSpec((1, d_, tb), lambda i, j: (i, 0, j)),
            pl.BlockSpec((_Q, _K, d_), lambda i, j: (0, 0, 0)),
            pl.BlockSpec((_Q, _K, d_), lambda i, j: (0, 0, 0)),
            pl.BlockSpec((3, _K, d_), lambda i, j: (0, 0, 0)),
            pl.BlockSpec((_Q, _K), lambda i, j: (0, 0)),
        ],
        out_specs=[
            pl.BlockSpec((1, d_, tb), lambda i, j: (i, 0, j)),
            pl.BlockSpec((1, 1, tb), lambda i, j: (i * g + j, 0, 0)),
            pl.BlockSpec((1, 1, tb), lambda i, j: (i * g + j, 0, 0)),
            pl.BlockSpec((1, _Q, 128), lambda i, j: (i * g + j, 0, 0)),
        ],
        out_shape=[
            jax.ShapeDtypeStruct((b, d_, t), jnp.float32),
            jax.ShapeDtypeStruct((b * g, 1, tb), jnp.int32),
            jax.ShapeDtypeStruct((b * g, 1, tb), jnp.int32),
            jax.ShapeDtypeStruct((b * g, _Q, 128), jnp.float32),
        ],
        compiler_params=pltpu.CompilerParams(
            dimension_semantics=("parallel", "parallel")),
        interpret=interpret,
    )(x, codebooks * 2.0, codebooks, _split3(codebooks[0]), cnorm)
    indices = jnp.stack([idx0.reshape(b, t), idx1.reshape(b, t)], axis=-1)
    loss = lossp[:, :, 0].sum(axis=0) / (b * t * d_)
    return out, indices, loss


def kernel(audio_input, image_input, audio_codebooks, image_codebooks):
    recon_audio, audio_indices, vq_audio_loss = _rvq(
        audio_input, audio_codebooks, 1024)

    bi, di, h, w = image_input.shape
    xi = image_input.reshape(bi, di, h * w)
    recon_img, image_indices, vq_image_loss = _rvq(xi, image_codebooks, 1024)
    recon_image = recon_img.reshape(bi, di, h, w)

    return (recon_audio, recon_image, vq_audio_loss, vq_image_loss,
            audio_indices, image_indices)


# speedup vs baseline: 1.0358x; 1.0358x over previous
import jax
import jax.numpy as jnp
from jax.experimental import pallas as pl
from jax.experimental.pallas import tpu as pltpu

_K = 1024
_D = 64
_Q = 2


def _dot(a, b, dims):
    return jax.lax.dot_general(a, b, (dims, ((), ())),
                               preferred_element_type=jnp.float32,
                               precision=jax.lax.Precision.DEFAULT)


def _rvq_kernel(x_ref, cb2_ref, cb_ref, sp_ref, cn_ref,
                out_ref, idx0_ref, idx1_ref, loss_ref):
    x = jnp.transpose(x_ref[0], (1, 0))
    tb = x.shape[0]
    lane_iota = jax.lax.broadcasted_iota(jnp.int32, (1, 128), 1)
    residual = x
    quant_out = jnp.zeros_like(x)
    losses = []
    idx_refs = (idx0_ref, idx1_ref)
    for q in range(_Q):
        cnorm = cn_ref[q]
        rnorm = jnp.sum(residual * residual, axis=1, keepdims=True)
        prod2 = _dot(residual, cb2_ref[q], ((1,), (1,)))
        d = (rnorm - prod2) + cnorm[None, :]
        dmin = jnp.min(d, axis=1, keepdims=True)
        key = None
        for c in range(_K // 128):
            dc = jax.lax.slice_in_dim(d, c * 128, (c + 1) * 128, axis=1)
            kc = jnp.where(dc == dmin, lane_iota + (128 * c), _K)
            key = kc if key is None else jnp.minimum(key, kc)
        idxc = jnp.min(key, axis=1, keepdims=True)
        idx_refs[q][0, 0] = idxc[:, 0]
        onehot = jnp.concatenate(
            [(lane_iota + (128 * c) == idxc).astype(jnp.float32)
             for c in range(_K // 128)], axis=1)
        if q == 0:
            quant = ((_dot(onehot, sp_ref[0], ((1,), (0,)))
                      + _dot(onehot, sp_ref[1], ((1,), (0,))))
                     + _dot(onehot, sp_ref[2], ((1,), (0,))))
        else:
            quant = _dot(onehot, cb_ref[q], ((1,), (0,)))
        diff = quant - residual
        losses.append(jnp.sum(diff * diff))
        quant_out = quant_out + (residual + (quant - residual))
        residual = residual - quant
    out_ref[0] = jnp.transpose(quant_out, (1, 0))
    loss_ref[0] = jnp.stack([jnp.broadcast_to(l, (128,)) for l in losses])


def _split3(cb):
    mask = jnp.int32(-65536)
    hi = jnp.bitwise_and(cb.view(jnp.int32), mask).view(jnp.float32)
    r = cb - hi
    mid = jnp.bitwise_and(r.view(jnp.int32), mask).view(jnp.float32)
    lo = r - mid
    return jnp.stack([hi, mid, lo])


def _rvq(x, codebooks, tb, interpret=False):
    b, d_, t = x.shape
    g = t // tb
    cnorm = jnp.sum(codebooks * codebooks, axis=-1)
    out, idx0, idx1, lossp = pl.pallas_call(
        _rvq_kernel,
        grid=(b, g),
        in_specs=[
            pl.BlockSpec((1, d_, tb), lambda i, j: (i, 0, j)),
            pl.BlockSpec((_Q, _K, d_), lambda i, j: (0, 0, 0)),
            pl.BlockSpec((_Q, _K, d_), lambda i, j: (0, 0, 0)),
            pl.BlockSpec((3, _K, d_), lambda i, j: (0, 0, 0)),
            pl.BlockSpec((_Q, _K), lambda i, j: (0, 0)),
        ],
        out_specs=[
            pl.BlockSpec((1, d_, tb), lambda i, j: (i, 0, j)),
            pl.BlockSpec((1, 1, tb), lambda i, j: (i * g + j, 0, 0)),
            pl.BlockSpec((1, 1, tb), lambda i, j: (i * g + j, 0, 0)),
            pl.BlockSpec((1, _Q, 128), lambda i, j: (i * g + j, 0, 0)),
        ],
        out_shape=[
            jax.ShapeDtypeStruct((b, d_, t), jnp.float32),
            jax.ShapeDtypeStruct((b * g, 1, tb), jnp.int32),
            jax.ShapeDtypeStruct((b * g, 1, tb), jnp.int32),
            jax.ShapeDtypeStruct((b * g, _Q, 128), jnp.float32),
        ],
        compiler_params=pltpu.CompilerParams(
            dimension_semantics=("parallel", "parallel")),
        interpret=interpret,
    )(x, codebooks * 2.0, codebooks, _split3(codebooks[0]), cnorm)
    indices = jnp.stack([idx0.reshape(b, t), idx1.reshape(b, t)], axis=-1)
    loss = lossp[:, :, 0].sum(axis=0) / (b * t * d_)
    return out, indices, loss


def kernel(audio_input, image_input, audio_codebooks, image_codebooks):
    recon_audio, audio_indices, vq_audio_loss = _rvq(
        audio_input, audio_codebooks, 2048)

    bi, di, h, w = image_input.shape
    xi = image_input.reshape(bi, di, h * w)
    recon_img, image_indices, vq_image_loss = _rvq(xi, image_codebooks, 1024)
    recon_image = recon_img.reshape(bi, di, h, w)

    return (recon_audio, recon_image, vq_audio_loss, vq_image_loss,
            audio_indices, image_indices)


# speedup vs baseline: 1.0913x; 1.0536x over previous
import jax
import jax.numpy as jnp
from jax.experimental import pallas as pl
from jax.experimental.pallas import tpu as pltpu

_K = 1024
_D = 64
_Q = 2


def _dot(a, b, dims):
    return jax.lax.dot_general(a, b, (dims, ((), ())),
                               preferred_element_type=jnp.float32,
                               precision=jax.lax.Precision.DEFAULT)


def _rvq_kernel(x_ref, cb2_ref, cb_ref, sp_ref, cn_ref,
                out_ref, idx0_ref, idx1_ref, loss_ref):
    x = jnp.transpose(x_ref[0], (1, 0))
    tb = x.shape[0]
    lane_iota = jax.lax.broadcasted_iota(
        jnp.int32, (1, 128), 1).astype(jnp.float32)
    residual = x
    quant_out = jnp.zeros_like(x)
    losses = []
    idx_refs = (idx0_ref, idx1_ref)
    for q in range(_Q):
        cnorm = cn_ref[q]
        rnorm = jnp.sum(residual * residual, axis=1, keepdims=True)
        prod2 = _dot(residual, cb2_ref[q], ((1,), (1,)))
        d = (rnorm - prod2) + cnorm[None, :]
        dmin = jnp.min(d, axis=1, keepdims=True)
        key = None
        for c in range(_K // 128):
            dc = jax.lax.slice_in_dim(d, c * 128, (c + 1) * 128, axis=1)
            kc = jnp.where(dc == dmin, lane_iota + (128.0 * c),
                           jnp.float32(_K))
            key = kc if key is None else jnp.minimum(key, kc)
        idxc = jnp.min(key, axis=1, keepdims=True)
        idx_refs[q][0, 0] = idxc[:, 0].astype(jnp.int32)
        onehot = jnp.concatenate(
            [(lane_iota + (128.0 * c) == idxc).astype(jnp.float32)
             for c in range(_K // 128)], axis=1)
        if q == 0:
            quant = ((_dot(onehot, sp_ref[0], ((1,), (0,)))
                      + _dot(onehot, sp_ref[1], ((1,), (0,))))
                     + _dot(onehot, sp_ref[2], ((1,), (0,))))
        else:
            quant = _dot(onehot, cb_ref[q], ((1,), (0,)))
        diff = quant - residual
        losses.append(jnp.sum(diff * diff))
        quant_out = quant_out + (residual + (quant - residual))
        residual = residual - quant
    out_ref[0] = jnp.transpose(quant_out, (1, 0))
    loss_ref[0] = jnp.stack([jnp.broadcast_to(l, (128,)) for l in losses])


def _split3(cb):
    mask = jnp.int32(-65536)
    hi = jnp.bitwise_and(cb.view(jnp.int32), mask).view(jnp.float32)
    r = cb - hi
    mid = jnp.bitwise_and(r.view(jnp.int32), mask).view(jnp.float32)
    lo = r - mid
    return jnp.stack([hi, mid, lo])


def _rvq(x, codebooks, tb, interpret=False):
    b, d_, t = x.shape
    g = t // tb
    cnorm = jnp.sum(codebooks * codebooks, axis=-1)
    out, idx0, idx1, lossp = pl.pallas_call(
        _rvq_kernel,
        grid=(b, g),
        in_specs=[
            pl.BlockSpec((1, d_, tb), lambda i, j: (i, 0, j)),
            pl.BlockSpec((_Q, _K, d_), lambda i, j: (0, 0, 0)),
            pl.BlockSpec((_Q, _K, d_), lambda i, j: (0, 0, 0)),
            pl.BlockSpec((3, _K, d_), lambda i, j: (0, 0, 0)),
            pl.BlockSpec((_Q, _K), lambda i, j: (0, 0)),
        ],
        out_specs=[
            pl.BlockSpec((1, d_, tb), lambda i, j: (i, 0, j)),
            pl.BlockSpec((1, 1, tb), lambda i, j: (i * g + j, 0, 0)),
            pl.BlockSpec((1, 1, tb), lambda i, j: (i * g + j, 0, 0)),
            pl.BlockSpec((1, _Q, 128), lambda i, j: (i * g + j, 0, 0)),
        ],
        out_shape=[
            jax.ShapeDtypeStruct((b, d_, t), jnp.float32),
            jax.ShapeDtypeStruct((b * g, 1, tb), jnp.int32),
            jax.ShapeDtypeStruct((b * g, 1, tb), jnp.int32),
            jax.ShapeDtypeStruct((b * g, _Q, 128), jnp.float32),
        ],
        compiler_params=pltpu.CompilerParams(
            dimension_semantics=("parallel", "parallel")),
        interpret=interpret,
    )(x, codebooks * 2.0, codebooks, _split3(codebooks[0]), cnorm)
    indices = jnp.stack([idx0.reshape(b, t), idx1.reshape(b, t)], axis=-1)
    loss = lossp[:, :, 0].sum(axis=0) / (b * t * d_)
    return out, indices, loss


def kernel(audio_input, image_input, audio_codebooks, image_codebooks):
    recon_audio, audio_indices, vq_audio_loss = _rvq(
        audio_input, audio_codebooks, 2048)

    bi, di, h, w = image_input.shape
    xi = image_input.reshape(bi, di, h * w)
    recon_img, image_indices, vq_image_loss = _rvq(xi, image_codebooks, 1024)
    recon_image = recon_img.reshape(bi, di, h, w)

    return (recon_audio, recon_image, vq_audio_loss, vq_image_loss,
            audio_indices, image_indices)


# speedup vs baseline: 1.4696x; 1.3466x over previous
import jax
import jax.numpy as jnp
from jax.experimental import pallas as pl
from jax.experimental.pallas import tpu as pltpu

_K = 1024
_D = 64
_Q = 2


def _dot(a, b, dims):
    return jax.lax.dot_general(a, b, (dims, ((), ())),
                               preferred_element_type=jnp.float32,
                               precision=jax.lax.Precision.DEFAULT)


def _rvq_kernel(x_ref, cb2_ref, cb_ref, sp_ref, cn_ref,
                out_ref, idx0_ref, idx1_ref, loss_ref):
    x = jnp.transpose(x_ref[0], (1, 0))
    tb = x.shape[0]
    lane_iota = jax.lax.broadcasted_iota(
        jnp.int32, (1, 128), 1).astype(jnp.float32)
    residual = x
    quant_out = jnp.zeros_like(x)
    losses = []
    idx_refs = (idx0_ref, idx1_ref)
    for q in range(_Q):
        cnorm = cn_ref[q]
        rnorm = jnp.sum(residual * residual, axis=1, keepdims=True)
        prod2 = _dot(residual, cb2_ref[q], ((1,), (1,)))
        d = (rnorm - prod2) + cnorm[None, :]
        dmin = jnp.min(d, axis=1, keepdims=True)
        key = None
        for c in range(_K // 128):
            dc = jax.lax.slice_in_dim(d, c * 128, (c + 1) * 128, axis=1)
            kc = jnp.where(dc == dmin, lane_iota + (128.0 * c),
                           jnp.float32(_K))
            key = kc if key is None else jnp.minimum(key, kc)
        idxc = jnp.min(key, axis=1, keepdims=True)
        idx_refs[q][0, 0] = idxc[:, 0].astype(jnp.int32)
        onehot = jnp.concatenate(
            [(lane_iota + (128.0 * c) == idxc).astype(jnp.float32)
             for c in range(_K // 128)], axis=1)
        if q == 0:
            g3 = _dot(onehot, sp_ref[...], ((1,), (0,)))
            quant = ((jax.lax.slice_in_dim(g3, 0, _D, axis=1)
                      + jax.lax.slice_in_dim(g3, _D, 2 * _D, axis=1))
                     + jax.lax.slice_in_dim(g3, 2 * _D, 3 * _D, axis=1))
        else:
            quant = _dot(onehot, cb_ref[q], ((1,), (0,)))
        diff = quant - residual
        losses.append(jnp.sum(diff * diff))
        quant_out = quant_out + (residual + (quant - residual))
        residual = residual - quant
    out_ref[0] = jnp.transpose(quant_out, (1, 0))
    loss_ref[0] = jnp.stack([jnp.broadcast_to(l, (128,)) for l in losses])


def _split3(cb):
    mask = jnp.int32(-65536)
    hi = jnp.bitwise_and(cb.view(jnp.int32), mask).view(jnp.float32)
    r = cb - hi
    mid = jnp.bitwise_and(r.view(jnp.int32), mask).view(jnp.float32)
    lo = r - mid
    return jnp.concatenate([hi, mid, lo], axis=-1)


def _rvq(x, codebooks, tb, interpret=False):
    b, d_, t = x.shape
    g = t // tb
    cnorm = jnp.sum(codebooks * codebooks, axis=-1)
    out, idx0, idx1, lossp = pl.pallas_call(
        _rvq_kernel,
        grid=(b, g),
        in_specs=[
            pl.BlockSpec((1, d_, tb), lambda i, j: (i, 0, j)),
            pl.BlockSpec((_Q, _K, d_), lambda i, j: (0, 0, 0)),
            pl.BlockSpec((_Q, _K, d_), lambda i, j: (0, 0, 0)),
            pl.BlockSpec((_K, 3 * d_), lambda i, j: (0, 0)),
            pl.BlockSpec((_Q, _K), lambda i, j: (0, 0)),
        ],
        out_specs=[
            pl.BlockSpec((1, d_, tb), lambda i, j: (i, 0, j)),
            pl.BlockSpec((1, 1, tb), lambda i, j: (i * g + j, 0, 0)),
            pl.BlockSpec((1, 1, tb), lambda i, j: (i * g + j, 0, 0)),
            pl.BlockSpec((1, _Q, 128), lambda i, j: (i * g + j, 0, 0)),
        ],
        out_shape=[
            jax.ShapeDtypeStruct((b, d_, t), jnp.float32),
            jax.ShapeDtypeStruct((b * g, 1, tb), jnp.int32),
            jax.ShapeDtypeStruct((b * g, 1, tb), jnp.int32),
            jax.ShapeDtypeStruct((b * g, _Q, 128), jnp.float32),
        ],
        compiler_params=pltpu.CompilerParams(
            dimension_semantics=("parallel", "parallel")),
        interpret=interpret,
    )(x, codebooks * 2.0, codebooks, _split3(codebooks[0]), cnorm)
    indices = jnp.stack([idx0.reshape(b, t), idx1.reshape(b, t)], axis=-1)
    loss = lossp[:, :, 0].sum(axis=0) / (b * t * d_)
    return out, indices, loss


def kernel(audio_input, image_input, audio_codebooks, image_codebooks):
    recon_audio, audio_indices, vq_audio_loss = _rvq(
        audio_input, audio_codebooks, 2048)

    bi, di, h, w = image_input.shape
    xi = image_input.reshape(bi, di, h * w)
    recon_img, image_indices, vq_image_loss = _rvq(xi, image_codebooks, 1024)
    recon_image = recon_img.reshape(bi, di, h, w)

    return (recon_audio, recon_image, vq_audio_loss, vq_image_loss,
            audio_indices, image_indices)
